# Initial kernel scaffold; baseline (speedup 1.0000x reference)
import functools

import jax
import jax.numpy as jnp
from jax import lax
from jax.experimental import pallas as pl
from jax.experimental.pallas import tpu as pltpu
from jax.experimental.pallas import tpu_sc as plsc

NC = 2
NS = 16
NW = NC * NS
L = 16

N_NODES = 10000
D = 256
G = 64
CHUNK = 312
SUB = 104
TAIL = N_NODES - NW * CHUNK
ACHUNK = 320
ATAIL = N_NODES - 31 * ACHUNK

_mesh = plsc.VectorSubcoreMesh(core_axis_name="c", subcore_axis_name="s")


def _zero_fill(ref, rows):
    z = jnp.zeros((L,), jnp.float32)
    for i in range(rows):
        for j in range(ref.shape[1] // L):
            ref[i, pl.ds(j * L, L)] = z


@functools.partial(
    pl.kernel,
    out_type=(
        jax.ShapeDtypeStruct((NC * G, D), jnp.float32),
        jax.ShapeDtypeStruct((NC * G, L), jnp.float32),
    ),
    mesh=_mesh,
    scratch_types=[
        pltpu.VMEM((CHUNK, D), jnp.float32),
        pltpu.VMEM((SUB,), jnp.int32),
        pltpu.VMEM((SUB,), jnp.int32),
        pltpu.VMEM((SUB,), jnp.int32),
        pltpu.VMEM((TAIL,), jnp.int32),
        pltpu.VMEM((SUB, L), jnp.float32),
        pltpu.VMEM((4, D), jnp.float32),
        pltpu.VMEM((4, L), jnp.float32),
        pltpu.VMEM_SHARED((G, D), jnp.float32),
        pltpu.VMEM_SHARED((G, L), jnp.float32),
    ],
)
def _partial_pool(x_hbm, batch_hbm, psums_hbm, pcnts_hbm,
                  x_v, i0, i1, i2, it, ones_v, zs, zc, s_sums, s_cnts):
    cid = lax.axis_index("c")
    sid = lax.axis_index("s")
    wid = sid * NC + cid

    _zero_fill(zs, 4)
    _zero_fill(zc, 4)
    pltpu.sync_copy(zs, s_sums.at[pl.ds(sid * 4, 4)])
    pltpu.sync_copy(zc, s_cnts.at[pl.ds(sid * 4, 4)])

    one = jnp.ones((L,), jnp.float32)
    for i in range(SUB):
        ones_v[i, :] = one

    plsc.subcore_barrier()

    base = wid * CHUNK
    pltpu.sync_copy(batch_hbm.at[pl.ds(base, SUB)], i0)
    pltpu.sync_copy(batch_hbm.at[pl.ds(base + SUB, SUB)], i1)
    pltpu.sync_copy(batch_hbm.at[pl.ds(base + 2 * SUB, SUB)], i2)
    pltpu.sync_copy(x_hbm.at[pl.ds(base, CHUNK)], x_v)

    for k, iref in enumerate((i0, i1, i2)):
        pltpu.sync_copy(x_v.at[pl.ds(k * SUB, SUB)], s_sums.at[iref], add=True)
        pltpu.sync_copy(ones_v, s_cnts.at[iref], add=True)

    @pl.when(wid == 0)
    def _():
        tbase = NW * CHUNK
        pltpu.sync_copy(batch_hbm.at[pl.ds(tbase, TAIL)], it)
        pltpu.sync_copy(x_hbm.at[pl.ds(tbase, TAIL)], x_v.at[pl.ds(0, TAIL)])
        pltpu.sync_copy(x_v.at[pl.ds(0, TAIL)], s_sums.at[it], add=True)
        pltpu.sync_copy(ones_v.at[pl.ds(0, TAIL)], s_cnts.at[it], add=True)

    plsc.subcore_barrier()

    @pl.when(sid == 0)
    def _():
        pltpu.sync_copy(s_sums, psums_hbm.at[pl.ds(cid * G, G)])
        pltpu.sync_copy(s_cnts, pcnts_hbm.at[pl.ds(cid * G, G)])


@functools.partial(
    pl.kernel,
    out_type=(
        jax.ShapeDtypeStruct((G, D), jnp.float32),
        jax.ShapeDtypeStruct((N_NODES,), jnp.float32),
    ),
    mesh=_mesh,
    scratch_types=[
        pltpu.VMEM((G, L), jnp.float32),
        pltpu.VMEM((G, L), jnp.float32),
        pltpu.VMEM((G, L), jnp.float32),
        pltpu.VMEM((2, D), jnp.float32),
        pltpu.VMEM((2, D), jnp.float32),
        pltpu.VMEM((2, D), jnp.float32),
        pltpu.VMEM((ACHUNK,), jnp.int32),
        pltpu.VMEM((ACHUNK,), jnp.float32),
    ],
)
def _finalize(psums_hbm, pcnts_hbm, batch_hbm, pooled_hbm, att_hbm,
              c0_v, c1_v, inv_v, a_v, b_v, p_v, bidx_v, att_v):
    cid = lax.axis_index("c")
    sid = lax.axis_index("s")
    wid = sid * NC + cid

    pltpu.sync_copy(pcnts_hbm.at[pl.ds(0, G)], c0_v)
    pltpu.sync_copy(pcnts_hbm.at[pl.ds(G, G)], c1_v)
    one = jnp.ones((L,), jnp.float32)
    for s in range(G):
        tot = c0_v[s, :] + c1_v[s, :]
        inv_v[s, :] = one / jnp.maximum(tot, one)

    row = 2 * wid
    pltpu.sync_copy(psums_hbm.at[pl.ds(row, 2)], a_v)
    pltpu.sync_copy(psums_hbm.at[pl.ds(G + row, 2)], b_v)
    for r in range(2):
        scale = plsc.load_gather(
            inv_v, [jnp.full((L,), row + r, jnp.int32), lax.iota(jnp.int32, L)])
        for j in range(D // L):
            s = a_v[r, pl.ds(j * L, L)] + b_v[r, pl.ds(j * L, L)]
            p_v[r, pl.ds(j * L, L)] = s * scale
    pltpu.sync_copy(p_v, pooled_hbm.at[pl.ds(row, 2)])

    zl = jnp.zeros((L,), jnp.int32)

    def fill(n):
        abase = wid * ACHUNK
        pltpu.sync_copy(batch_hbm.at[pl.ds(abase, n)], bidx_v.at[pl.ds(0, n)])
        for g in range(n // L):
            idx = bidx_v[pl.ds(g * L, L)]
            att_v[pl.ds(g * L, L)] = plsc.load_gather(inv_v, [idx, zl])
        pltpu.sync_copy(att_v.at[pl.ds(0, n)], att_hbm.at[pl.ds(abase, n)])

    @pl.when(wid < NW - 1)
    def _():
        fill(ACHUNK)

    @pl.when(wid == NW - 1)
    def _():
        fill(ATAIL)


def kernel(x, edge_index, batch):
    psums, pcnts = _partial_pool(x, batch)
    x_pooled, attention = _finalize(psums, pcnts, batch)
    return (x_pooled, attention)


# speedup vs baseline: 2.2769x; 2.2769x over previous
import functools

import jax
import jax.numpy as jnp
from jax import lax
from jax.experimental import pallas as pl
from jax.experimental.pallas import tpu as pltpu
from jax.experimental.pallas import tpu_sc as plsc

NC = 2
NS = 16
NW = NC * NS
L = 16

N_NODES = 10000
D = 256
G = 64
CHUNK = 320
NSUB = 4
SROWS = CHUNK // NSUB
TAIL = N_NODES - (NW - 1) * CHUNK
STRIPE = G // NS
DJ = D // L
CL = 128

_mesh = plsc.VectorSubcoreMesh(core_axis_name="c", subcore_axis_name="s")


@functools.partial(
    pl.kernel,
    out_type=(
        jax.ShapeDtypeStruct((NC * G, D), jnp.float32),
        jax.ShapeDtypeStruct((NC * G, CL), jnp.float32),
    ),
    mesh=_mesh,
    compiler_params=pltpu.CompilerParams(needs_layout_passes=False),
    scratch_types=[
        pltpu.VMEM((SROWS, D), jnp.float32),
        pltpu.VMEM((SROWS, D), jnp.float32),
        pltpu.VMEM((CHUNK,), jnp.int32),
        pltpu.VMEM((G, D), jnp.float32),
        pltpu.VMEM((G, CL), jnp.float32),
        pltpu.VMEM((STRIPE, D), jnp.float32),
        pltpu.VMEM((STRIPE, D), jnp.float32),
        pltpu.VMEM((STRIPE, CL), jnp.float32),
        pltpu.VMEM((STRIPE, CL), jnp.float32),
        pltpu.VMEM_SHARED((NS * G, D), jnp.float32),
        pltpu.VMEM_SHARED((NS * G, CL), jnp.float32),
        pltpu.SemaphoreType.DMA,
        pltpu.SemaphoreType.DMA,
    ],
)
def _partial_pool(x_hbm, batch_hbm, psums_hbm, pcnts_hbm,
                  xb0, xb1, bidx_v, lsum, lcnt,
                  acc, tmp, cacc, ctmp, s_slots, s_cslots, sem0, sem1):
    cid = lax.axis_index("c")
    sid = lax.axis_index("s")
    wid = sid * NC + cid

    zv = jnp.zeros((L,), jnp.float32)
    for i in range(G):
        for j in range(CL // L):
            lcnt[i, pl.ds(j * L, L)] = zv
        for j in range(DJ):
            lsum[i, pl.ds(j * L, L)] = zv

    lanes = lax.iota(jnp.int32, L)
    ones = jnp.ones((L,), jnp.float32)
    base = wid * CHUNK

    def accumulate(buf, goff, ngroups):
        def body(g, carry):
            segs = bidx_v[pl.ds(g * L, L)]
            plsc.addupdate_scatter(lcnt, [segs, lanes], ones)
            for r in range(L):
                rowidx = jnp.full((L,), segs[r], jnp.int32)
                row = (g - goff) * L + r
                for j in range(DJ):
                    vals = buf[row, pl.ds(j * L, L)]
                    plsc.addupdate_scatter(
                        lsum, [rowidx, lanes + (j * L)], vals)
            return carry
        lax.fori_loop(goff, goff + ngroups, body, 0)

    ng_sub = SROWS // L

    @pl.when(wid < NW - 1)
    def _():
        pltpu.sync_copy(batch_hbm.at[pl.ds(base, CHUNK)], bidx_v)
        bufs = (xb0, xb1)
        sems = (sem0, sem1)
        cps = [None, None]
        cps[0] = pltpu.async_copy(x_hbm.at[pl.ds(base, SROWS)], xb0, sem0)
        for sub in range(NSUB):
            if sub + 1 < NSUB:
                cps[(sub + 1) % 2] = pltpu.async_copy(
                    x_hbm.at[pl.ds(base + (sub + 1) * SROWS, SROWS)],
                    bufs[(sub + 1) % 2], sems[(sub + 1) % 2])
            cps[sub % 2].wait()
            accumulate(bufs[sub % 2], sub * ng_sub, ng_sub)

    @pl.when(wid == NW - 1)
    def _():
        pltpu.sync_copy(batch_hbm.at[pl.ds(base, TAIL)],
                        bidx_v.at[pl.ds(0, TAIL)])
        pltpu.sync_copy(x_hbm.at[pl.ds(base, TAIL)], xb0)
        accumulate(xb0, 0, TAIL // L)

    pltpu.sync_copy(lsum, s_slots.at[pl.ds(sid * G, G)])
    pltpu.sync_copy(lcnt, s_cslots.at[pl.ds(sid * G, G)])
    plsc.subcore_barrier()

    rbase = sid * STRIPE
    pltpu.sync_copy(s_slots.at[pl.ds(rbase, STRIPE)], acc)
    pltpu.sync_copy(s_cslots.at[pl.ds(rbase, STRIPE)], cacc)

    def merge(t, carry):
        pltpu.sync_copy(s_slots.at[pl.ds(t * G + rbase, STRIPE)], tmp)
        pltpu.sync_copy(s_cslots.at[pl.ds(t * G + rbase, STRIPE)], ctmp)
        for i in range(STRIPE):
            for j in range(CL // L):
                cacc[i, pl.ds(j * L, L)] = (
                    cacc[i, pl.ds(j * L, L)] + ctmp[i, pl.ds(j * L, L)])
            for j in range(DJ):
                acc[i, pl.ds(j * L, L)] = (
                    acc[i, pl.ds(j * L, L)] + tmp[i, pl.ds(j * L, L)])
        return carry
    lax.fori_loop(1, NS, merge, 0)

    pltpu.sync_copy(acc, psums_hbm.at[pl.ds(cid * G + rbase, STRIPE)])
    pltpu.sync_copy(cacc, pcnts_hbm.at[pl.ds(cid * G + rbase, STRIPE)])


@functools.partial(
    pl.kernel,
    out_type=(
        jax.ShapeDtypeStruct((G, D), jnp.float32),
        jax.ShapeDtypeStruct((N_NODES,), jnp.float32),
    ),
    mesh=_mesh,
    compiler_params=pltpu.CompilerParams(needs_layout_passes=False),
    scratch_types=[
        pltpu.VMEM((G, CL), jnp.float32),
        pltpu.VMEM((G, CL), jnp.float32),
        pltpu.VMEM((G, L), jnp.float32),
        pltpu.VMEM((2, D), jnp.float32),
        pltpu.VMEM((2, D), jnp.float32),
        pltpu.VMEM((2, D), jnp.float32),
        pltpu.VMEM((CHUNK,), jnp.int32),
        pltpu.VMEM((CHUNK,), jnp.float32),
    ],
)
def _finalize(psums_hbm, pcnts_hbm, batch_hbm, pooled_hbm, att_hbm,
              c0_v, c1_v, inv_v, a_v, b_v, p_v, bidx_v, att_v):
    cid = lax.axis_index("c")
    sid = lax.axis_index("s")
    wid = sid * NC + cid

    pltpu.sync_copy(pcnts_hbm.at[pl.ds(0, G)], c0_v)
    pltpu.sync_copy(pcnts_hbm.at[pl.ds(G, G)], c1_v)
    one = jnp.ones((L,), jnp.float32)
    for s in range(G):
        tot16 = c0_v[s, pl.ds(0, L)] + c1_v[s, pl.ds(0, L)]
        tot = jnp.full((L,), jnp.sum(tot16), jnp.float32)
        inv_v[s, :] = one / jnp.maximum(tot, one)

    row = 2 * wid
    pltpu.sync_copy(psums_hbm.at[pl.ds(row, 2)], a_v)
    pltpu.sync_copy(psums_hbm.at[pl.ds(G + row, 2)], b_v)
    for r in range(2):
        scale = plsc.load_gather(
            inv_v, [jnp.full((L,), row + r, jnp.int32), lax.iota(jnp.int32, L)])
        for j in range(DJ):
            s = a_v[r, pl.ds(j * L, L)] + b_v[r, pl.ds(j * L, L)]
            p_v[r, pl.ds(j * L, L)] = s * scale
    pltpu.sync_copy(p_v, pooled_hbm.at[pl.ds(row, 2)])

    zl = jnp.zeros((L,), jnp.int32)

    def fill(n):
        abase = wid * CHUNK
        pltpu.sync_copy(batch_hbm.at[pl.ds(abase, n)], bidx_v.at[pl.ds(0, n)])
        for g in range(n // L):
            idx = bidx_v[pl.ds(g * L, L)]
            att_v[pl.ds(g * L, L)] = plsc.load_gather(inv_v, [idx, zl])
        pltpu.sync_copy(att_v.at[pl.ds(0, n)], att_hbm.at[pl.ds(abase, n)])

    @pl.when(wid < NW - 1)
    def _():
        fill(CHUNK)

    @pl.when(wid == NW - 1)
    def _():
        fill(TAIL)


def kernel(x, edge_index, batch):
    psums, pcnts = _partial_pool(x, batch)
    x_pooled, attention = _finalize(psums, pcnts, batch)
    return (x_pooled, attention)
